# Initial kernel scaffold; baseline (speedup 1.0000x reference)
import jax
import jax.numpy as jnp
from jax.experimental import pallas as pl


def kernel(x2d, projected_pix, scale_2d, fov_mask, pix_z, depth_img):
    raise NotImplementedError("write your pallas kernel here")


# speedup vs baseline: 2.9095x; 2.9095x over previous
import functools

import jax
import jax.numpy as jnp
from jax import lax
from jax.experimental import pallas as pl
from jax.experimental.pallas import tpu as pltpu
from jax.experimental.pallas import tpu_sc as plsc

_SCENE = (256, 256, 32)
_PS = 2

_C, _H, _W = 128, 96, 320
_HW = _H * _W
_N = (_SCENE[0] // _PS) * (_SCENE[1] // _PS) * (_SCENE[2] // _PS)

_NWORKERS = 32
_SL = _N // _NWORKERS
_CPW = _C // _NWORKERS
_CH = 1024
_NCH = _N // _CH

_mesh = plsc.VectorSubcoreMesh(core_axis_name="c", subcore_axis_name="s")
_params = pltpu.CompilerParams(needs_layout_passes=False)


@functools.partial(
    pl.kernel,
    out_type=[
        jax.ShapeDtypeStruct((_N,), jnp.int32),
        jax.ShapeDtypeStruct((_N,), jnp.float32),
    ],
    mesh=_mesh,
    compiler_params=_params,
    scratch_types=[
        pltpu.VMEM((_HW,), jnp.float32),
        pltpu.VMEM((_SL,), jnp.int32),
        pltpu.VMEM((_SL,), jnp.int32),
        pltpu.VMEM((_SL,), jnp.int32),
        pltpu.VMEM((_SL,), jnp.int32),
        pltpu.VMEM((_SL,), jnp.float32),
        pltpu.VMEM((_SL,), jnp.float32),
        pltpu.VMEM((_SL,), jnp.int32),
        pltpu.VMEM((_SL,), jnp.float32),
    ],
)
def _idx_weight(xs_hbm, ys_hbm, xss_hbm, yss_hbm, fov_hbm, pz_hbm, depth_hbm,
                idx_hbm, w_hbm,
                depth_v, xs_v, ys_v, xss_v, yss_v, fov_v, pz_v, idx_v, w_v):
    wid = lax.axis_index("s") * 2 + lax.axis_index("c")
    base = wid * _SL
    pltpu.sync_copy(depth_hbm, depth_v)
    pltpu.sync_copy(xs_hbm.at[pl.ds(base, _SL)], xs_v)
    pltpu.sync_copy(ys_hbm.at[pl.ds(base, _SL)], ys_v)
    pltpu.sync_copy(xss_hbm.at[pl.ds(base, _SL)], xss_v)
    pltpu.sync_copy(yss_hbm.at[pl.ds(base, _SL)], yss_v)
    pltpu.sync_copy(fov_hbm.at[pl.ds(base, _SL)], fov_v)
    pltpu.sync_copy(pz_hbm.at[pl.ds(base, _SL)], pz_v)

    def body(j, carry):
        o = j * 16
        x = xs_v[pl.ds(o, 16)]
        y = ys_v[pl.ds(o, 16)]
        di = y * _W + x
        xs_ = xss_v[pl.ds(o, 16)]
        ys_ = yss_v[pl.ds(o, 16)]
        idx_v[pl.ds(o, 16)] = ys_ * _W + xs_
        d = plsc.load_gather(depth_v, [di])
        t = pz_v[pl.ds(o, 16)] - d
        wgt = jnp.exp(t * t * -2.0)
        wgt = jnp.where(d == 0.0, jnp.float32(1.0), wgt)
        w_v[pl.ds(o, 16)] = wgt * fov_v[pl.ds(o, 16)]
        return carry

    lax.fori_loop(0, _SL // 16, body, 0)
    pltpu.sync_copy(idx_v, idx_hbm.at[pl.ds(base, _SL)])
    pltpu.sync_copy(w_v, w_hbm.at[pl.ds(base, _SL)])


@functools.partial(
    pl.kernel,
    out_type=jax.ShapeDtypeStruct((_C, _N), jnp.float32),
    mesh=_mesh,
    compiler_params=_params,
    scratch_types=[
        pltpu.VMEM((_CPW, _HW), jnp.float32),
        pltpu.VMEM((_CH,), jnp.int32),
        pltpu.VMEM((_CH,), jnp.float32),
        pltpu.VMEM((_CPW, _CH), jnp.float32),
    ],
)
def _gather_scale(src_hbm, idx_hbm, w_hbm, out_hbm, rows_v, idx_v, w_v, out_v):
    wid = lax.axis_index("s") * 2 + lax.axis_index("c")
    c0 = wid * _CPW
    pltpu.sync_copy(src_hbm.at[pl.ds(c0, _CPW)], rows_v)

    def chunk(k, carry):
        n0 = k * _CH
        pltpu.sync_copy(idx_hbm.at[pl.ds(n0, _CH)], idx_v)
        pltpu.sync_copy(w_hbm.at[pl.ds(n0, _CH)], w_v)

        def grp(j, c2):
            o = j * 16
            iv = idx_v[pl.ds(o, 16)]
            wv = w_v[pl.ds(o, 16)]
            for c in range(_CPW):
                cv = jnp.full((16,), c, jnp.int32)
                g = plsc.load_gather(rows_v, [cv, iv])
                out_v[c, pl.ds(o, 16)] = g * wv
            return c2

        lax.fori_loop(0, _CH // 16, grp, 0)
        pltpu.sync_copy(out_v, out_hbm.at[pl.ds(c0, _CPW), pl.ds(n0, _CH)])
        return carry

    lax.fori_loop(0, _NCH, chunk, 0)


def kernel(x2d, projected_pix, scale_2d, fov_mask, pix_z, depth_img):
    c, h, w = x2d.shape
    xs = projected_pix[:, 0]
    ys = projected_pix[:, 1]
    xss = xs // scale_2d
    yss = ys // scale_2d
    fov_f = fov_mask.astype(jnp.float32)
    pz = pix_z.reshape(-1)
    depth_flat = depth_img.reshape(-1)
    src = x2d.reshape(c, h * w)

    idx, wgt = _idx_weight(xs, ys, xss, yss, fov_f, pz, depth_flat)
    out = _gather_scale(src, idx, wgt)
    return out.reshape(c, _SCENE[0] // _PS, _SCENE[1] // _PS, _SCENE[2] // _PS)


# speedup vs baseline: 3.4256x; 1.1774x over previous
import functools

import jax
import jax.numpy as jnp
from jax import lax
from jax.experimental import pallas as pl
from jax.experimental.pallas import tpu as pltpu
from jax.experimental.pallas import tpu_sc as plsc

_SCENE = (256, 256, 32)
_PS = 2

_C, _H, _W = 128, 96, 320
_HW = _H * _W
_N = (_SCENE[0] // _PS) * (_SCENE[1] // _PS) * (_SCENE[2] // _PS)

_NWORKERS = 32
_SL = _N // _NWORKERS
_CPW = _C // _NWORKERS
_CH = 512
_NCH = _N // _CH

_mesh = plsc.VectorSubcoreMesh(core_axis_name="c", subcore_axis_name="s")
_params = pltpu.CompilerParams(needs_layout_passes=False)


@functools.partial(
    pl.kernel,
    out_type=[
        jax.ShapeDtypeStruct((_N,), jnp.int32),
        jax.ShapeDtypeStruct((_N,), jnp.float32),
    ],
    mesh=_mesh,
    compiler_params=_params,
    scratch_types=[
        pltpu.VMEM((_HW,), jnp.float32),
        pltpu.VMEM((2 * _SL,), jnp.int32),
        pltpu.VMEM((16,), jnp.int32),
        pltpu.VMEM((_SL,), jnp.float32),
        pltpu.VMEM((_SL,), jnp.float32),
        pltpu.VMEM((_SL,), jnp.int32),
        pltpu.VMEM((_SL,), jnp.float32),
    ],
)
def _idx_weight(pix_hbm, scale_hbm, fov_hbm, pz_hbm, depth_hbm,
                idx_hbm, w_hbm,
                depth_v, pix_v, scale_v, fov_v, pz_v, idx_v, w_v):
    wid = lax.axis_index("s") * 2 + lax.axis_index("c")
    base = wid * _SL
    pltpu.sync_copy(depth_hbm, depth_v)
    pltpu.sync_copy(pix_hbm.at[pl.ds(2 * base, 2 * _SL)], pix_v)
    pltpu.sync_copy(scale_hbm, scale_v)
    pltpu.sync_copy(fov_hbm.at[pl.ds(base, _SL)], fov_v)
    pltpu.sync_copy(pz_hbm.at[pl.ds(base, _SL)], pz_v)

    iota = lax.iota(jnp.int32, 16)
    scale = scale_v[...]

    def body(j, carry):
        o = j * 16
        p2 = 2 * o + 2 * iota
        x = plsc.load_gather(pix_v, [p2])
        y = plsc.load_gather(pix_v, [p2 + 1])
        di = y * _W + x
        xs_ = lax.div(x, scale)
        ys_ = lax.div(y, scale)
        idx_v[pl.ds(o, 16)] = ys_ * _W + xs_
        d = plsc.load_gather(depth_v, [di])
        t = pz_v[pl.ds(o, 16)] - d
        wgt = jnp.exp(t * t * -2.0)
        wgt = jnp.where(d == 0.0, jnp.float32(1.0), wgt)
        w_v[pl.ds(o, 16)] = wgt * fov_v[pl.ds(o, 16)]
        return carry

    lax.fori_loop(0, _SL // 16, body, 0)
    pltpu.sync_copy(idx_v, idx_hbm.at[pl.ds(base, _SL)])
    pltpu.sync_copy(w_v, w_hbm.at[pl.ds(base, _SL)])


@functools.partial(
    pl.kernel,
    out_type=jax.ShapeDtypeStruct((_C, _N), jnp.float32),
    mesh=_mesh,
    compiler_params=_params,
    scratch_types=[
        pltpu.VMEM((_CPW, _HW), jnp.float32),
        pltpu.VMEM((2, _CH), jnp.int32),
        pltpu.VMEM((2, _CH), jnp.float32),
        pltpu.VMEM((2, _CPW, _CH), jnp.float32),
        pltpu.SemaphoreType.DMA,
        pltpu.SemaphoreType.DMA,
        pltpu.SemaphoreType.DMA,
        pltpu.SemaphoreType.DMA,
    ],
)
def _gather_scale(src_hbm, idx_hbm, w_hbm, out_hbm,
                  rows_v, idx2, w2, out2, sin0, sin1, sout0, sout1):
    wid = lax.axis_index("s") * 2 + lax.axis_index("c")
    c0 = wid * _CPW
    sins = (sin0, sin1)
    souts = (sout0, sout1)

    def start_in(k, b):
        n0 = k * _CH
        pltpu.async_copy(idx_hbm.at[pl.ds(n0, _CH)], idx2.at[b], sins[b])
        pltpu.async_copy(w_hbm.at[pl.ds(n0, _CH)], w2.at[b], sins[b])

    def wait_in(k, b):
        n0 = k * _CH
        pltpu.make_async_copy(idx_hbm.at[pl.ds(n0, _CH)], idx2.at[b], sins[b]).wait()
        pltpu.make_async_copy(w_hbm.at[pl.ds(n0, _CH)], w2.at[b], sins[b]).wait()

    def out_copy(k, b):
        n0 = k * _CH
        return pltpu.make_async_copy(
            out2.at[b], out_hbm.at[pl.ds(c0, _CPW), pl.ds(n0, _CH)], souts[b])

    start_in(0, 0)
    start_in(1, 1)
    pltpu.sync_copy(src_hbm.at[pl.ds(c0, _CPW)], rows_v)

    def step(i, carry):
        for b in range(2):
            k = 2 * i + b
            wait_in(k, b)

            @pl.when(i >= 1)
            def _():
                out_copy(k - 2, b).wait()

            def grp(j, c2):
                o = j * 16
                iv = idx2[b, pl.ds(o, 16)]
                wv = w2[b, pl.ds(o, 16)]
                for c in range(_CPW):
                    cv = jnp.full((16,), c, jnp.int32)
                    g = plsc.load_gather(rows_v, [cv, iv])
                    out2[b, c, pl.ds(o, 16)] = g * wv
                return c2

            lax.fori_loop(0, _CH // 16, grp, 0)
            out_copy(k, b).start()

            @pl.when(i < _NCH // 2 - 1)
            def _():
                start_in(k + 2, b)
        return carry

    lax.fori_loop(0, _NCH // 2, step, 0)
    out_copy(_NCH - 2, 0).wait()
    out_copy(_NCH - 1, 1).wait()


def kernel(x2d, projected_pix, scale_2d, fov_mask, pix_z, depth_img):
    c, h, w = x2d.shape
    pix_flat = projected_pix.reshape(-1)
    scale_vec = jnp.full((16,), scale_2d, jnp.int32)
    fov_f = fov_mask.astype(jnp.float32)
    pz = pix_z.reshape(-1)
    depth_flat = depth_img.reshape(-1)
    src = x2d.reshape(c, h * w)

    idx, wgt = _idx_weight(pix_flat, scale_vec, fov_f, pz, depth_flat)
    out = _gather_scale(src, idx, wgt)
    return out.reshape(c, _SCENE[0] // _PS, _SCENE[1] // _PS, _SCENE[2] // _PS)
